# baseline (speedup 1.0000x reference)
import functools

import jax
import jax.numpy as jnp
from jax import lax
from jax.experimental import pallas as pl
from jax.experimental.pallas import tpu as pltpu
from jax.experimental.pallas import tpu_sc as plsc

_B, _T, _V = 16, 32, 100000
_N = _B * _T


def _sc_gather(seqs_flat, probs_flat):
    mesh = plsc.VectorSubcoreMesh(core_axis_name="c", subcore_axis_name="s")
    num_cores = mesh.num_cores
    num_workers = num_cores * mesh.num_subcores
    per = _N // num_workers

    @functools.partial(
        pl.kernel,
        out_type=jax.ShapeDtypeStruct((_N,), jnp.float32),
        mesh=mesh,
        scratch_types=[
            pltpu.VMEM((per,), jnp.int32),
            pltpu.VMEM((per,), jnp.int32),
            pltpu.VMEM((per,), jnp.float32),
            pltpu.SemaphoreType.DMA,
        ],
    )
    def gather_kernel(seqs_hbm, probs_hbm, out_hbm, seq_v, idx_v, val_v, sem):
        wid = lax.axis_index("s") * num_cores + lax.axis_index("c")
        base = wid * per
        pltpu.sync_copy(seqs_hbm.at[pl.ds(base, per)], seq_v)
        rows = base + lax.iota(jnp.int32, per)
        idx_v[...] = rows * _V + seq_v[...]
        pltpu.async_copy(probs_hbm.at[idx_v], val_v, sem).wait()
        pltpu.sync_copy(val_v, out_hbm.at[pl.ds(base, per)])

    return gather_kernel(seqs_flat, probs_flat)


def _tc_finish_body(p_ref, s_ref, r_ref, b_ref, o_ref):
    pv = p_ref[...]
    sv = s_ref[...]
    seq_len = jnp.sum((sv > 0).astype(jnp.int32), axis=1, keepdims=True) + 1
    tt = lax.broadcasted_iota(jnp.int32, (_B, _T), 1)
    maskv = (tt < seq_len).astype(jnp.float32)
    adv = r_ref[...] - b_ref[...]
    losses = -jnp.log(pv + 1e-10) * adv
    o_ref[0, 0] = jnp.sum(losses * maskv) / jnp.sum(maskv)


def _tc_finish(p, seqs, reward, baseline):
    return pl.pallas_call(
        _tc_finish_body,
        out_shape=jax.ShapeDtypeStruct((1, 1), jnp.float32),
        out_specs=pl.BlockSpec(memory_space=pltpu.SMEM),
    )(p, seqs, reward, baseline)


def kernel(reward, baseline, probs, seqs):
    seqs = seqs.astype(jnp.int32)
    p = _sc_gather(seqs.reshape(-1), probs.reshape(-1))
    out = _tc_finish(p.reshape(_B, _T), seqs, reward.reshape(_B, 1),
                     baseline.reshape(_B, 1))
    return out[0, 0]


# speedup vs baseline: 11.8062x; 11.8062x over previous
import dataclasses
import functools

import jax
import jax.numpy as jnp
from jax import lax
from jax.experimental import pallas as pl
from jax.experimental.pallas import tpu as pltpu
from jax.experimental.pallas import tpu_sc as plsc

_B, _T, _V = 16, 32, 100000
_N = _B * _T
_HALF = _T // 2
_WMAX = _V - 128


def _sc_gather(seqs, probs, reward, baseline):
    mesh = plsc.VectorSubcoreMesh(core_axis_name="c", subcore_axis_name="s")
    num_cores = mesh.num_cores
    f32 = jnp.float32
    cp = pltpu.CompilerParams()
    if "needs_layout_passes" in pltpu.CompilerParams.__dataclass_fields__:
        cp = dataclasses.replace(cp, needs_layout_passes=False)

    @functools.partial(
        pl.kernel,
        compiler_params=cp,
        out_type=(
            jax.ShapeDtypeStruct((_N,), f32),
            jax.ShapeDtypeStruct((_N,), f32),
            jax.ShapeDtypeStruct((_N,), f32),
        ),
        mesh=mesh,
        scratch_types=[
            pltpu.VMEM((_B, _T), jnp.int32),
            pltpu.VMEM((16,), f32),
            pltpu.VMEM((16,), f32),
            pltpu.VMEM((_HALF, 8, 128), f32),
            pltpu.VMEM((_HALF,), f32),
            pltpu.VMEM((_HALF,), f32),
            pltpu.VMEM((_HALF,), f32),
            pltpu.SemaphoreType.DMA,
        ],
    )
    def gather_kernel(seqs_hbm, probs_hbm, rew_hbm, base_hbm,
                      p_hbm, w_hbm, m_hbm,
                      seq_all, rew_v, base_v, buf, p_v, w_v, m_v, sem):
        wid = lax.axis_index("s") * num_cores + lax.axis_index("c")
        b = wid // 2
        t0 = (wid % 2) * _HALF
        base = wid * _HALF
        pltpu.sync_copy(seqs_hbm, seq_all)
        pltpu.sync_copy(rew_hbm, rew_v)
        pltpu.sync_copy(base_hbm, base_v)

        iota = lax.iota(jnp.int32, _HALF)
        seq_v = seq_all[b, pl.ds(t0, _HALF)]

        copies = []
        for j in range(_HALF):
            sj = jnp.sum(jnp.where(iota == j, seq_v, 0))
            wj = pl.multiple_of((sj >> 7) << 7, 128)
            t8 = t0 + 8 * (j // 8)
            copies.append(
                pltpu.async_copy(
                    probs_hbm.at[b, pl.ds(t8, 8), pl.ds(wj, 128)],
                    buf.at[j], sem,
                )
            )

        row_lo = seq_all[b, pl.ds(0, _HALF)]
        row_hi = seq_all[b, pl.ds(_HALF, _HALF)]
        cnt = (jnp.sum(jnp.where(row_lo > 0, 1, 0))
               + jnp.sum(jnp.where(row_hi > 0, 1, 0)))
        mask = ((t0 + iota) < (cnt + 1)).astype(f32)
        adv_all = rew_v[...] - base_v[...]
        adv_b = jnp.sum(jnp.where(lax.iota(jnp.int32, 16) == b, adv_all, 0.0))
        m_v[...] = mask
        w_v[...] = mask * adv_b

        for c in copies:
            c.wait()
        lane = seq_v & 127
        row = iota & 7
        p_v[...] = plsc.load_gather(buf, [iota, row, lane])

        pltpu.sync_copy(p_v, p_hbm.at[pl.ds(base, _HALF)])
        pltpu.sync_copy(w_v, w_hbm.at[pl.ds(base, _HALF)])
        pltpu.sync_copy(m_v, m_hbm.at[pl.ds(base, _HALF)])

    return gather_kernel(seqs, probs, reward, baseline)


def _tc_finish_body(p_ref, w_ref, m_ref, o_ref):
    losses = -jnp.log(p_ref[...] + 1e-10) * w_ref[...]
    o_ref[0, 0] = jnp.sum(losses) / jnp.sum(m_ref[...])


def _tc_finish(p, w, m):
    return pl.pallas_call(
        _tc_finish_body,
        out_shape=jax.ShapeDtypeStruct((1, 1), jnp.float32),
        out_specs=pl.BlockSpec(memory_space=pltpu.SMEM),
    )(p, w, m)


def kernel(reward, baseline, probs, seqs):
    seqs = seqs.astype(jnp.int32)
    p, w, m = _sc_gather(seqs, probs, reward, baseline)
    out = _tc_finish(p, w, m)
    return out[0, 0]


# speedup vs baseline: 12.2028x; 1.0336x over previous
import dataclasses
import functools

import jax
import jax.numpy as jnp
from jax import lax
from jax.experimental import pallas as pl
from jax.experimental.pallas import tpu as pltpu
from jax.experimental.pallas import tpu_sc as plsc

_B, _T, _V = 16, 32, 100000
_N = _B * _T
_HALF = _T // 2


def _sc_gather(seqs, probs):
    mesh = plsc.VectorSubcoreMesh(core_axis_name="c", subcore_axis_name="s")
    num_cores = mesh.num_cores
    cp = pltpu.CompilerParams()
    if "needs_layout_passes" in pltpu.CompilerParams.__dataclass_fields__:
        cp = dataclasses.replace(cp, needs_layout_passes=False)

    @functools.partial(
        pl.kernel,
        compiler_params=cp,
        out_type=jax.ShapeDtypeStruct((_N,), jnp.float32),
        mesh=mesh,
        scratch_types=[
            pltpu.VMEM((_B, _T), jnp.int32),
            pltpu.VMEM((_HALF, 8, 128), jnp.float32),
            pltpu.VMEM((_HALF,), jnp.float32),
            pltpu.SemaphoreType.DMA,
            pltpu.SemaphoreType.DMA,
        ],
    )
    def gather_kernel(seqs_hbm, probs_hbm, p_hbm, seq_all, buf, p_v, sem,
                      sem2):
        wid = lax.axis_index("s") * num_cores + lax.axis_index("c")
        b = wid // 2
        t0 = (wid % 2) * _HALF
        pltpu.async_copy(seqs_hbm, seq_all, sem2).wait()
        iota = lax.iota(jnp.int32, _HALF)
        seq_v = seq_all[b, pl.ds(t0, _HALF)]

        @pl.loop(0, _HALF)
        def _(j):
            sj = jnp.sum(jnp.where(iota == j, seq_v, 0))
            wj = pl.multiple_of((sj >> 7) << 7, 128)
            t8 = pl.multiple_of(t0 + ((j >> 3) << 3), 8)
            pltpu.async_copy(probs_hbm.at[b, pl.ds(t8, 8), pl.ds(wj, 128)],
                             buf.at[j], sem)

        pltpu.make_async_copy(
            probs_hbm.at[pl.ds(0, _HALF), pl.ds(0, 8), pl.ds(0, 128)],
            buf, sem,
        ).wait()

        lane = seq_v & 127
        row = iota & 7
        p_v[...] = plsc.load_gather(buf, [iota, row, lane])
        pltpu.sync_copy(p_v, p_hbm.at[pl.ds(wid * _HALF, _HALF)])

    return gather_kernel(seqs, probs)


def _tc_finish_body(p_ref, s_ref, r_ref, b_ref, o_ref):
    pv = p_ref[...]
    sv = s_ref[...]
    seq_len = jnp.sum((sv > 0).astype(jnp.int32), axis=1, keepdims=True) + 1
    tt = lax.broadcasted_iota(jnp.int32, (_B, _T), 1)
    maskv = (tt < seq_len).astype(jnp.float32)
    adv = r_ref[...] - b_ref[...]
    losses = -jnp.log(pv + 1e-10) * adv
    o_ref[0, 0] = jnp.sum(losses * maskv) / jnp.sum(maskv)


def _tc_finish(p, seqs, reward, baseline):
    return pl.pallas_call(
        _tc_finish_body,
        out_shape=jax.ShapeDtypeStruct((1, 1), jnp.float32),
        out_specs=pl.BlockSpec(memory_space=pltpu.SMEM),
    )(p, seqs, reward, baseline)


def kernel(reward, baseline, probs, seqs):
    seqs = seqs.astype(jnp.int32)
    p = _sc_gather(seqs, probs)
    out = _tc_finish(p.reshape(_B, _T), seqs, reward.reshape(_B, 1),
                     baseline.reshape(_B, 1))
    return out[0, 0]


# speedup vs baseline: 29.5101x; 2.4183x over previous
import jax
import jax.numpy as jnp
from jax import lax
from jax.experimental import pallas as pl
from jax.experimental.pallas import tpu as pltpu

_B, _T, _V = 16, 32, 100000
_N = _B * _T


def _body(s_smem, r_smem, b_smem, probs_hbm, oh_ref, s_vmem, o_ref, buf, sem):
    def issue(i, _):
        b = i >> 5
        t = i & 31
        s = s_smem[b, t]
        w = pl.multiple_of((s >> 7) << 7, 128)
        pltpu.make_async_copy(
            probs_hbm.at[b, t, pl.ds(w, 128)], buf.at[b, t], sem
        ).start()
        return _

    lax.fori_loop(0, _N, issue, None)
    pltpu.make_async_copy(
        probs_hbm.at[pl.ds(0, _B), pl.ds(0, _T), pl.ds(0, 128)], buf, sem
    ).wait()

    pv = jnp.sum(jnp.where(oh_ref[...] > 0.0, buf[...], 0.0), axis=2)
    sv = s_vmem[...]
    seq_len = jnp.sum((sv > 0).astype(jnp.int32), axis=1, keepdims=True) + 1
    tt = lax.broadcasted_iota(jnp.int32, (_B, _T), 1)
    maskv = (tt < seq_len).astype(jnp.float32)
    lrows = -jnp.log(pv + 1e-10) * maskv
    acc = 0.0
    for b in range(_B):
        acc += (r_smem[b] - b_smem[b]) * jnp.sum(lrows[b])
    o_ref[0, 0] = acc / jnp.sum(maskv)


def kernel(reward, baseline, probs, seqs):
    seqs = seqs.astype(jnp.int32)
    oh = (lax.broadcasted_iota(jnp.int32, (_B, _T, 128), 2)
          == (seqs & 127)[:, :, None]).astype(jnp.float32)
    out = pl.pallas_call(
        _body,
        in_specs=[
            pl.BlockSpec(memory_space=pltpu.MemorySpace.SMEM),
            pl.BlockSpec(memory_space=pltpu.MemorySpace.SMEM),
            pl.BlockSpec(memory_space=pltpu.MemorySpace.SMEM),
            pl.BlockSpec(memory_space=pltpu.MemorySpace.HBM),
            pl.BlockSpec(memory_space=pltpu.MemorySpace.VMEM),
            pl.BlockSpec(memory_space=pltpu.MemorySpace.VMEM),
        ],
        out_specs=pl.BlockSpec(memory_space=pltpu.MemorySpace.SMEM),
        out_shape=jax.ShapeDtypeStruct((1, 1), jnp.float32),
        scratch_shapes=[
            pltpu.VMEM((_B, _T, 128), jnp.float32),
            pltpu.SemaphoreType.DMA,
        ],
    )(seqs, reward, baseline, probs, oh, seqs)
    return out[0, 0]


# speedup vs baseline: 41.2836x; 1.3990x over previous
import jax
import jax.numpy as jnp
from jax import lax
from jax.experimental import pallas as pl
from jax.experimental.pallas import tpu as pltpu

_B, _T, _V = 16, 32, 100000
_N = _B * _T


def _body(w_smem, r_smem, b_smem, probs_hbm, oh_ref, s_vmem, o_ref, buf, sem):
    def issue(b, carry):
        for t in range(_T):
            w = pl.multiple_of(w_smem[b, t], 128)
            pltpu.make_async_copy(
                probs_hbm.at[b, t, pl.ds(w, 128)], buf.at[b, t], sem
            ).start()
        return carry

    lax.fori_loop(0, _B, issue, None)
    pltpu.make_async_copy(
        probs_hbm.at[pl.ds(0, _B), pl.ds(0, _T), pl.ds(0, 128)], buf, sem
    ).wait()

    pv = jnp.sum(jnp.where(oh_ref[...] > 0.0, buf[...], 0.0), axis=2)
    sv = s_vmem[...]
    seq_len = jnp.sum((sv > 0).astype(jnp.int32), axis=1, keepdims=True) + 1
    tt = lax.broadcasted_iota(jnp.int32, (_B, _T), 1)
    maskv = (tt < seq_len).astype(jnp.float32)
    lrows = -jnp.log(pv + 1e-10) * maskv
    acc = 0.0
    for b in range(_B):
        acc += (r_smem[b] - b_smem[b]) * jnp.sum(lrows[b])
    o_ref[0, 0] = acc / jnp.sum(maskv)


def kernel(reward, baseline, probs, seqs):
    seqs = seqs.astype(jnp.int32)
    w_arr = (seqs >> 7) << 7
    oh = (lax.broadcasted_iota(jnp.int32, (_B, _T, 128), 2)
          == (seqs & 127)[:, :, None]).astype(jnp.float32)
    out = pl.pallas_call(
        _body,
        in_specs=[
            pl.BlockSpec(memory_space=pltpu.MemorySpace.SMEM),
            pl.BlockSpec(memory_space=pltpu.MemorySpace.SMEM),
            pl.BlockSpec(memory_space=pltpu.MemorySpace.SMEM),
            pl.BlockSpec(memory_space=pltpu.MemorySpace.HBM),
            pl.BlockSpec(memory_space=pltpu.MemorySpace.VMEM),
            pl.BlockSpec(memory_space=pltpu.MemorySpace.VMEM),
        ],
        out_specs=pl.BlockSpec(memory_space=pltpu.MemorySpace.SMEM),
        out_shape=jax.ShapeDtypeStruct((1, 1), jnp.float32),
        scratch_shapes=[
            pltpu.VMEM((_B, _T, 128), jnp.float32),
            pltpu.SemaphoreType.DMA,
        ],
    )(w_arr, reward, baseline, probs, oh, seqs)
    return out[0, 0]


# speedup vs baseline: 55.3188x; 1.3400x over previous
import jax
import jax.numpy as jnp
from jax import lax
from jax.experimental import pallas as pl
from jax.experimental.pallas import tpu as pltpu

_B, _T, _V = 16, 32, 100000
_N = _B * _T


def _body(s_smem, r_smem, b_smem, probs_hbm, s_vmem, o_ref, buf, sem):
    def issue(b, carry):
        for t in range(_T):
            w = pl.multiple_of((s_smem[b, t] >> 7) << 7, 128)
            pltpu.make_async_copy(
                probs_hbm.at[b, t, pl.ds(w, 128)], buf.at[b, t], sem
            ).start()
        return carry

    lax.fori_loop(0, _B, issue, None)
    pltpu.make_async_copy(
        probs_hbm.at[pl.ds(0, _B), pl.ds(0, _T), pl.ds(0, 128)], buf, sem
    ).wait()

    sv = s_vmem[...]
    lane3 = jnp.broadcast_to((sv & 127)[:, :, None], (_B, _T, 128))
    oh = lax.broadcasted_iota(jnp.int32, (_B, _T, 128), 2) == lane3
    pv = jnp.sum(jnp.where(oh, buf[...], 0.0), axis=2)
    seq_len = jnp.sum((sv > 0).astype(jnp.int32), axis=1, keepdims=True) + 1
    tt = lax.broadcasted_iota(jnp.int32, (_B, _T), 1)
    maskv = (tt < seq_len).astype(jnp.float32)
    lrows = -jnp.log(pv + 1e-10) * maskv
    acc = 0.0
    for b in range(_B):
        acc += (r_smem[b] - b_smem[b]) * jnp.sum(lrows[b])
    o_ref[0, 0] = acc / jnp.sum(maskv)


def kernel(reward, baseline, probs, seqs):
    seqs = seqs.astype(jnp.int32)
    out = pl.pallas_call(
        _body,
        in_specs=[
            pl.BlockSpec(memory_space=pltpu.MemorySpace.SMEM),
            pl.BlockSpec(memory_space=pltpu.MemorySpace.SMEM),
            pl.BlockSpec(memory_space=pltpu.MemorySpace.SMEM),
            pl.BlockSpec(memory_space=pltpu.MemorySpace.HBM),
            pl.BlockSpec(memory_space=pltpu.MemorySpace.VMEM),
        ],
        out_specs=pl.BlockSpec(memory_space=pltpu.MemorySpace.SMEM),
        out_shape=jax.ShapeDtypeStruct((1, 1), jnp.float32),
        scratch_shapes=[
            pltpu.VMEM((_B, _T, 128), jnp.float32),
            pltpu.SemaphoreType.DMA,
        ],
    )(seqs, reward, baseline, probs, seqs)
    return out[0, 0]


# speedup vs baseline: 56.5445x; 1.0222x over previous
import jax
import jax.numpy as jnp
from jax import lax
from jax.experimental import pallas as pl
from jax.experimental.pallas import tpu as pltpu

_B, _T, _V = 16, 32, 100000
_N = _B * _T


def _body(s_smem, r_smem, b_smem, probs_hbm, s_vmem, o_ref, buf, sem):
    def issue(b, carry):
        for t in range(_T):
            w = pl.multiple_of((s_smem[b, t] >> 7) << 7, 128)
            pltpu.make_async_copy(
                probs_hbm.at[b, t, pl.ds(w, 128)], buf.at[b, t], sem
            ).start()
        return carry

    lax.fori_loop(0, _B, issue, None)

    sv = s_vmem[...]
    lane3 = jnp.broadcast_to((sv & 127)[:, :, None], (_B, _T, 128))
    oh = lax.broadcasted_iota(jnp.int32, (_B, _T, 128), 2) == lane3
    seq_len = jnp.sum((sv > 0).astype(jnp.int32), axis=1, keepdims=True) + 1
    tt = lax.broadcasted_iota(jnp.int32, (_B, _T), 1)
    maskv = (tt < seq_len).astype(jnp.float32)

    pltpu.make_async_copy(
        probs_hbm.at[pl.ds(0, _B), pl.ds(0, _T), pl.ds(0, 128)], buf, sem
    ).wait()

    pv = jnp.sum(jnp.where(oh, buf[...], 0.0), axis=2)
    lrows = -jnp.log(pv + 1e-10) * maskv
    acc = 0.0
    for b in range(_B):
        acc += (r_smem[b] - b_smem[b]) * jnp.sum(lrows[b])
    o_ref[0, 0] = acc / jnp.sum(maskv)


def kernel(reward, baseline, probs, seqs):
    seqs = seqs.astype(jnp.int32)
    out = pl.pallas_call(
        _body,
        in_specs=[
            pl.BlockSpec(memory_space=pltpu.MemorySpace.SMEM),
            pl.BlockSpec(memory_space=pltpu.MemorySpace.SMEM),
            pl.BlockSpec(memory_space=pltpu.MemorySpace.SMEM),
            pl.BlockSpec(memory_space=pltpu.MemorySpace.HBM),
            pl.BlockSpec(memory_space=pltpu.MemorySpace.VMEM),
        ],
        out_specs=pl.BlockSpec(memory_space=pltpu.MemorySpace.SMEM),
        out_shape=jax.ShapeDtypeStruct((1, 1), jnp.float32),
        scratch_shapes=[
            pltpu.VMEM((_B, _T, 128), jnp.float32),
            pltpu.SemaphoreType.DMA,
        ],
    )(seqs, reward, baseline, probs, seqs)
    return out[0, 0]


# speedup vs baseline: 58.3340x; 1.0316x over previous
import jax
import jax.numpy as jnp
from jax import lax
from jax.experimental import pallas as pl
from jax.experimental.pallas import tpu as pltpu

_B, _T, _V = 16, 32, 100000
_N = _B * _T


def _body(s_smem, r_smem, b_smem, probs_hbm, s_vmem, o_ref, buf, sem):
    for b in range(_B):
        for t in range(_T):
            w = pl.multiple_of((s_smem[b, t] >> 7) << 7, 128)
            pltpu.make_async_copy(
                probs_hbm.at[b, t, pl.ds(w, 128)], buf.at[b, t], sem
            ).start()

    sv = s_vmem[...]
    lane3 = jnp.broadcast_to((sv & 127)[:, :, None], (_B, _T, 128))
    oh = lax.broadcasted_iota(jnp.int32, (_B, _T, 128), 2) == lane3
    seq_len = jnp.sum((sv > 0).astype(jnp.int32), axis=1, keepdims=True) + 1
    tt = lax.broadcasted_iota(jnp.int32, (_B, _T), 1)
    maskv = (tt < seq_len).astype(jnp.float32)

    pltpu.make_async_copy(
        probs_hbm.at[pl.ds(0, _B), pl.ds(0, _T), pl.ds(0, 128)], buf, sem
    ).wait()

    pv = jnp.sum(jnp.where(oh, buf[...], 0.0), axis=2)
    lrows = -jnp.log(pv + 1e-10) * maskv
    acc = 0.0
    for b in range(_B):
        acc += (r_smem[b] - b_smem[b]) * jnp.sum(lrows[b])
    o_ref[0, 0] = acc / jnp.sum(maskv)


def kernel(reward, baseline, probs, seqs):
    seqs = seqs.astype(jnp.int32)
    out = pl.pallas_call(
        _body,
        in_specs=[
            pl.BlockSpec(memory_space=pltpu.MemorySpace.SMEM),
            pl.BlockSpec(memory_space=pltpu.MemorySpace.SMEM),
            pl.BlockSpec(memory_space=pltpu.MemorySpace.SMEM),
            pl.BlockSpec(memory_space=pltpu.MemorySpace.HBM),
            pl.BlockSpec(memory_space=pltpu.MemorySpace.VMEM),
        ],
        out_specs=pl.BlockSpec(memory_space=pltpu.MemorySpace.SMEM),
        out_shape=jax.ShapeDtypeStruct((1, 1), jnp.float32),
        scratch_shapes=[
            pltpu.VMEM((_B, _T, 128), jnp.float32),
            pltpu.SemaphoreType.DMA,
        ],
    )(seqs, reward, baseline, probs, seqs)
    return out[0, 0]


# speedup vs baseline: 66.8550x; 1.1461x over previous
import jax
import jax.numpy as jnp
from jax import lax
from jax.experimental import pallas as pl
from jax.experimental.pallas import tpu as pltpu

_B, _T, _V = 16, 32, 100000
_G = 4
_NG = _B // _G


def _body(s_smem, r_smem, b_smem, probs_hbm, s_vmem, o_ref, buf, pv_ref, sems):
    for b in range(_B):
        for t in range(_T):
            w = pl.multiple_of((s_smem[b, t] >> 7) << 7, 128)
            pltpu.make_async_copy(
                probs_hbm.at[b, t, pl.ds(w, 128)], buf.at[b, t],
                sems.at[b // _G],
            ).start()

    sv = s_vmem[...]
    lane3 = jnp.broadcast_to((sv & 127)[:, :, None], (_B, _T, 128))
    oh = lax.broadcasted_iota(jnp.int32, (_B, _T, 128), 2) == lane3
    seq_len = jnp.sum((sv > 0).astype(jnp.int32), axis=1, keepdims=True) + 1
    tt = lax.broadcasted_iota(jnp.int32, (_B, _T), 1)
    maskv = (tt < seq_len).astype(jnp.float32)

    for g in range(_NG):
        rows = pl.ds(g * _G, _G)
        pltpu.make_async_copy(
            probs_hbm.at[rows, pl.ds(0, _T), pl.ds(0, 128)],
            buf.at[rows], sems.at[g],
        ).wait()
        pv_ref[rows] = jnp.sum(
            jnp.where(oh[g * _G:(g + 1) * _G], buf[g * _G:(g + 1) * _G], 0.0),
            axis=2,
        )

    lrows = -jnp.log(pv_ref[...] + 1e-10) * maskv
    acc = 0.0
    for b in range(_B):
        acc += (r_smem[b] - b_smem[b]) * jnp.sum(lrows[b])
    o_ref[0, 0] = acc / jnp.sum(maskv)


def kernel(reward, baseline, probs, seqs):
    seqs = seqs.astype(jnp.int32)
    out = pl.pallas_call(
        _body,
        in_specs=[
            pl.BlockSpec(memory_space=pltpu.MemorySpace.SMEM),
            pl.BlockSpec(memory_space=pltpu.MemorySpace.SMEM),
            pl.BlockSpec(memory_space=pltpu.MemorySpace.SMEM),
            pl.BlockSpec(memory_space=pltpu.MemorySpace.HBM),
            pl.BlockSpec(memory_space=pltpu.MemorySpace.VMEM),
        ],
        out_specs=pl.BlockSpec(memory_space=pltpu.MemorySpace.SMEM),
        out_shape=jax.ShapeDtypeStruct((1, 1), jnp.float32),
        scratch_shapes=[
            pltpu.VMEM((_B, _T, 128), jnp.float32),
            pltpu.VMEM((_B, _T), jnp.float32),
            pltpu.SemaphoreType.DMA((_NG,)),
        ],
    )(seqs, reward, baseline, probs, seqs)
    return out[0, 0]


# speedup vs baseline: 67.1009x; 1.0037x over previous
import jax
import jax.numpy as jnp
from jax import lax
from jax.experimental import pallas as pl
from jax.experimental.pallas import tpu as pltpu

_B, _T, _V = 16, 32, 100000
_G = 2
_NG = _B // _G


def _body(s_smem, r_smem, b_smem, probs_hbm, s_vmem, o_ref, buf, pv_ref, sems):
    for b in range(_B):
        for t in range(_T):
            w = pl.multiple_of((s_smem[b, t] >> 7) << 7, 128)
            pltpu.make_async_copy(
                probs_hbm.at[b, t, pl.ds(w, 128)], buf.at[b, t],
                sems.at[b // _G],
            ).start()

    sv = s_vmem[...]
    lane3 = jnp.broadcast_to((sv & 127)[:, :, None], (_B, _T, 128))
    oh = lax.broadcasted_iota(jnp.int32, (_B, _T, 128), 2) == lane3
    seq_len = jnp.sum((sv > 0).astype(jnp.int32), axis=1, keepdims=True) + 1
    tt = lax.broadcasted_iota(jnp.int32, (_B, _T), 1)
    maskv = (tt < seq_len).astype(jnp.float32)

    for g in range(_NG):
        rows = pl.ds(g * _G, _G)
        pltpu.make_async_copy(
            probs_hbm.at[rows, pl.ds(0, _T), pl.ds(0, 128)],
            buf.at[rows], sems.at[g],
        ).wait()
        pv_ref[rows] = jnp.sum(
            jnp.where(oh[g * _G:(g + 1) * _G], buf[g * _G:(g + 1) * _G], 0.0),
            axis=2,
        )

    lrows = -jnp.log(pv_ref[...] + 1e-10) * maskv
    acc = 0.0
    for b in range(_B):
        acc += (r_smem[b] - b_smem[b]) * jnp.sum(lrows[b])
    o_ref[0, 0] = acc / jnp.sum(maskv)


def kernel(reward, baseline, probs, seqs):
    seqs = seqs.astype(jnp.int32)
    out = pl.pallas_call(
        _body,
        in_specs=[
            pl.BlockSpec(memory_space=pltpu.MemorySpace.SMEM),
            pl.BlockSpec(memory_space=pltpu.MemorySpace.SMEM),
            pl.BlockSpec(memory_space=pltpu.MemorySpace.SMEM),
            pl.BlockSpec(memory_space=pltpu.MemorySpace.HBM),
            pl.BlockSpec(memory_space=pltpu.MemorySpace.VMEM),
        ],
        out_specs=pl.BlockSpec(memory_space=pltpu.MemorySpace.SMEM),
        out_shape=jax.ShapeDtypeStruct((1, 1), jnp.float32),
        scratch_shapes=[
            pltpu.VMEM((_B, _T, 128), jnp.float32),
            pltpu.VMEM((_B, _T), jnp.float32),
            pltpu.SemaphoreType.DMA((_NG,)),
        ],
    )(seqs, reward, baseline, probs, seqs)
    return out[0, 0]
